# baseline (speedup 1.0000x reference)
import functools

import jax
import jax.numpy as jnp
from jax import lax
from jax.experimental import pallas as pl
from jax.experimental.pallas import tpu as pltpu
from jax.experimental.pallas import tpu_sc as plsc

N_VERTEX = 50000
NU = 8
D_LAT = 128
N_ANCIL = 8
ROWS = N_VERTEX * NU
CHUNK = 400
NBUF = 2
PAIR = NBUF * CHUNK
NPAIR = ROWS // PAIR


def _sc_gather(table, idx_flat):
    info = plsc.get_sparse_core_info()
    nc, ns = info.num_cores, info.num_subcores
    nw = nc * ns
    pmax = -(-NPAIR // nw)
    jmax = -(-pmax // 2)

    mesh = plsc.VectorSubcoreMesh(core_axis_name="c", subcore_axis_name="s")

    @functools.partial(
        pl.kernel,
        out_type=jax.ShapeDtypeStruct((ROWS, D_LAT), jnp.float32),
        mesh=mesh,
        scratch_types=(
            [pltpu.VMEM((PAIR,), jnp.int32) for _ in range(2)]
            + [pltpu.VMEM((CHUNK, D_LAT), jnp.float32) for _ in range(NBUF)]
            + [pltpu.SemaphoreType.DMA for _ in range(2 * NBUF)]
        ),
    )
    def gather_kernel(table_hbm, idx_hbm, out_hbm, *bufs):
        wid = lax.axis_index("s") * nc + lax.axis_index("c")
        idx_v = bufs[:2]
        rows_v = bufs[2 : 2 + NBUF]
        gsem = bufs[2 + NBUF : 2 + 2 * NBUF]
        wsem = bufs[2 + 2 * NBUF : 2 + 3 * NBUF]

        def body(j, carry):
            for par in range(2):
                p = (j * 2 + par) * nw + wid

                @pl.when(p < NPAIR)
                def _(par=par, p=p):
                    pltpu.sync_copy(
                        idx_hbm.at[pl.ds(p * PAIR, PAIR)], idx_v[par]
                    )

                    for b in range(NBUF):
                        if par == 0:

                            @pl.when(j > 0)
                            def _(b=b):
                                pltpu.make_async_copy(
                                    rows_v[b],
                                    out_hbm.at[pl.ds(0, CHUNK)],
                                    wsem[b],
                                ).wait()

                        else:
                            pltpu.make_async_copy(
                                rows_v[b], out_hbm.at[pl.ds(0, CHUNK)], wsem[b]
                            ).wait()
                        ids = idx_v[par].at[pl.ds(b * CHUNK, CHUNK)]
                        pltpu.async_copy(table_hbm.at[ids], rows_v[b], gsem[b])

                    for b in range(NBUF):
                        ids = idx_v[par].at[pl.ds(b * CHUNK, CHUNK)]
                        pltpu.make_async_copy(
                            table_hbm.at[ids], rows_v[b], gsem[b]
                        ).wait()
                        pltpu.async_copy(
                            rows_v[b],
                            out_hbm.at[pl.ds((p * NBUF + b) * CHUNK, CHUNK)],
                            wsem[b],
                        )

            return carry

        lax.fori_loop(0, jmax, body, 0)

        for b in range(NBUF):
            pltpu.make_async_copy(
                rows_v[b], out_hbm.at[pl.ds(0, CHUNK)], wsem[b]
            ).wait()

    return gather_kernel(table, idx_flat)


def _tc_transpose(x):
    def tkernel(x_ref, o_ref):
        o_ref[...] = x_ref[...].T

    return pl.pallas_call(
        tkernel,
        out_shape=jax.ShapeDtypeStruct((N_VERTEX, N_ANCIL), jnp.float32),
    )(x)


def kernel(z_prime, x_ancil, index):
    idx_flat = index.reshape(ROWS).astype(jnp.int32)
    z_rows = _sc_gather(z_prime, idx_flat)
    z_tilde = z_rows.reshape(N_VERTEX, NU, D_LAT)
    x_ancil_tilde = _tc_transpose(x_ancil)
    return z_tilde, x_ancil_tilde


# speedup vs baseline: 1.0264x; 1.0264x over previous
import functools

import jax
import jax.numpy as jnp
from jax import lax
from jax.experimental import pallas as pl
from jax.experimental.pallas import tpu as pltpu
from jax.experimental.pallas import tpu_sc as plsc

N_VERTEX = 50000
NU = 8
D_LAT = 128
N_ANCIL = 8
ROWS = N_VERTEX * NU
CHUNK = 320
NCHUNK = ROWS // CHUNK
NBUF = 3


def _sc_gather(table, idx_flat):
    info = plsc.get_sparse_core_info()
    nc, ns = info.num_cores, info.num_subcores
    nw = nc * ns
    kmax = -(-NCHUNK // nw)
    jmax = -(-kmax // NBUF)

    mesh = plsc.VectorSubcoreMesh(core_axis_name="c", subcore_axis_name="s")

    @functools.partial(
        pl.kernel,
        out_type=jax.ShapeDtypeStruct((ROWS, D_LAT), jnp.float32),
        mesh=mesh,
        scratch_types=(
            [pltpu.VMEM((CHUNK,), jnp.int32) for _ in range(NBUF)]
            + [pltpu.VMEM((CHUNK, D_LAT), jnp.float32) for _ in range(NBUF)]
            + [pltpu.SemaphoreType.DMA for _ in range(2 * NBUF)]
        ),
    )
    def gather_kernel(table_hbm, idx_hbm, out_hbm, *bufs):
        wid = lax.axis_index("s") * nc + lax.axis_index("c")
        idx_v = bufs[:NBUF]
        rows_v = bufs[NBUF : 2 * NBUF]
        gsem = bufs[2 * NBUF : 3 * NBUF]
        wsem = bufs[3 * NBUF : 4 * NBUF]

        def body(j, carry):
            for b in range(NBUF):
                c = (j * NBUF + b) * nw + wid

                @pl.when(c < NCHUNK)
                def _(b=b, c=c):
                    @pl.when(j > 0)
                    def _():
                        pltpu.make_async_copy(
                            rows_v[b], out_hbm.at[pl.ds(0, CHUNK)], wsem[b]
                        ).wait()

                    base = c * CHUNK
                    pltpu.sync_copy(idx_hbm.at[pl.ds(base, CHUNK)], idx_v[b])
                    pltpu.async_copy(table_hbm.at[idx_v[b]], rows_v[b], gsem[b])

            for b in range(NBUF):
                c = (j * NBUF + b) * nw + wid

                @pl.when(c < NCHUNK)
                def _(b=b, c=c):
                    pltpu.make_async_copy(
                        table_hbm.at[idx_v[b]], rows_v[b], gsem[b]
                    ).wait()
                    pltpu.async_copy(
                        rows_v[b], out_hbm.at[pl.ds(c * CHUNK, CHUNK)], wsem[b]
                    )

            return carry

        lax.fori_loop(0, jmax, body, 0)

        for b in range(NBUF):
            pltpu.make_async_copy(
                rows_v[b], out_hbm.at[pl.ds(0, CHUNK)], wsem[b]
            ).wait()

    return gather_kernel(table, idx_flat)


def _tc_transpose(x):
    def tkernel(x_ref, o_ref):
        o_ref[...] = x_ref[...].T

    return pl.pallas_call(
        tkernel,
        out_shape=jax.ShapeDtypeStruct((N_VERTEX, N_ANCIL), jnp.float32),
    )(x)


def kernel(z_prime, x_ancil, index):
    idx_flat = index.reshape(ROWS).astype(jnp.int32)
    z_rows = _sc_gather(z_prime, idx_flat)
    z_tilde = z_rows.reshape(N_VERTEX, NU, D_LAT)
    x_ancil_tilde = _tc_transpose(x_ancil)
    return z_tilde, x_ancil_tilde
